# Initial kernel scaffold; baseline (speedup 1.0000x reference)
import jax
import jax.numpy as jnp
from jax.experimental import pallas as pl


def kernel(x, embedding):
    raise NotImplementedError("write your pallas kernel here")


# speedup vs baseline: 4.8095x; 4.8095x over previous
import functools

import jax
import jax.numpy as jnp
from jax import lax
from jax.experimental import pallas as pl
from jax.experimental.pallas import tpu as pltpu
from jax.experimental.pallas import tpu_sc as plsc

IW = 128
CHUNK = 8


@functools.lru_cache(maxsize=None)
def _build(N, D, NC, NS):
    NW = NC * NS
    R = N // IW
    RW = R // NW
    steps = RW // CHUNK

    mesh = plsc.VectorSubcoreMesh(core_axis_name="c", subcore_axis_name="s")

    @functools.partial(
        pl.kernel,
        mesh=mesh,
        out_type=jax.ShapeDtypeStruct((N, D), jnp.float32),
        scratch_types=[
            pltpu.VMEM((CHUNK, IW), jnp.int32),
            pltpu.VMEM((CHUNK * IW, D), jnp.float32),
            pltpu.SemaphoreType.DMA,
        ],
        compiler_params=pltpu.CompilerParams(use_tc_tiling_on_sc=False),
    )
    def emb_kernel(idx_hbm, table_hbm, out_hbm, idx_v, rows_v, sem):
        wid = lax.axis_index("s") * NC + lax.axis_index("c")
        row0 = wid * RW

        def step(g, carry):
            r = row0 + g * CHUNK
            pltpu.sync_copy(idx_hbm.at[pl.ds(r, CHUNK)], idx_v)
            copies = [
                pltpu.async_copy(
                    table_hbm.at[idx_v.at[j]],
                    rows_v.at[pl.ds(j * IW, IW)],
                    sem,
                )
                for j in range(CHUNK)
            ]
            for cpy in copies:
                cpy.wait()
            pltpu.sync_copy(rows_v, out_hbm.at[pl.ds(r * IW, CHUNK * IW)])
            return carry

        lax.fori_loop(0, steps, step, 0)

    return emb_kernel


def kernel(x, embedding):
    B, H = x.shape
    V, D = embedding.shape
    N = B * H
    info = plsc.get_sparse_core_info()
    NC, NS = info.num_cores, info.num_subcores
    idx = x.reshape(N // IW, IW).astype(jnp.int32)
    out = _build(N, D, NC, NS)(idx, embedding)
    return out.reshape(B, H, D)
